# Initial kernel scaffold; baseline (speedup 1.0000x reference)
import jax
import jax.numpy as jnp
from jax.experimental import pallas as pl


def kernel(z_hyp, leaf_emb, leaf_node_ids, threshold):
    raise NotImplementedError("write your pallas kernel here")


# speedup vs baseline: 4.7627x; 4.7627x over previous
import functools

import jax
import jax.numpy as jnp
from jax import lax
from jax.experimental import pallas as pl
from jax.experimental.pallas import tpu as pltpu
from jax.experimental.pallas import tpu_sc as plsc

EPS = 1e-7
B = 1024
L = 100000
D = 64
LB = 2048
NBLK = 49
LP = NBLK * LB
GPS = LB // 128
NSLOT = NBLK * 128
K = 5
NCAND = K * GPS
BIGV = 1e30
BIGF = 3e38
BIGI = 2 ** 30


def _k1_body(z_ref, y_ref, bm_ref):
    i = pl.program_id(0)
    z = z_ref[...]
    y = y_ref[...]
    x2 = jnp.sum(z * z, axis=1)
    y2 = jnp.sum(y * y, axis=1)
    r = 1.0 / (1.0 - y2)
    col = jax.lax.broadcasted_iota(jnp.int32, (1, LB), 1)[0] + i * LB
    u = jnp.where(col < L, 0.0, BIGV)
    d = lax.dot_general(z, y, (((1,), (1,)), ((), ())),
                        preferred_element_type=jnp.float32)
    sq = x2[:, None] + y2[None, :] - 2.0 * d
    p = sq * r[None, :] + u[None, :]
    m = p[:, 0:128]
    for g in range(1, GPS):
        m = jnp.minimum(m, p[:, g * 128:(g + 1) * 128])
    bm_ref[...] = m


def _stage1(z, leaf_pad):
    return pl.pallas_call(
        _k1_body,
        grid=(NBLK,),
        in_specs=[
            pl.BlockSpec((B, D), lambda i: (0, 0)),
            pl.BlockSpec((LB, D), lambda i: (i, 0)),
        ],
        out_specs=pl.BlockSpec((B, 128), lambda i: (0, i)),
        out_shape=jax.ShapeDtypeStruct((B, NSLOT), jnp.float32),
    )(z, leaf_pad)


NCHUNK = 7
CW = NSLOT // NCHUNK


def _k2_body(bm_ref, cols_ref, sv_ref, si_ref):
    c = pl.program_id(0)
    v = bm_ref[...]
    sl = jax.lax.broadcasted_iota(jnp.int32, (B, CW), 1) + c * CW
    for k in range(K):
        m = jnp.min(v, axis=1)
        eq = v == m[:, None]
        idx = jnp.min(jnp.where(eq, sl, BIGI), axis=1)
        sv_ref[c, k, :] = m
        si_ref[c, k, :] = idx
        v = jnp.where(sl == idx[:, None], BIGF, v)

    @pl.when(c == NCHUNK - 1)
    def _():
        sv = sv_ref[...].reshape(NCHUNK * K, B)
        si = si_ref[...].reshape(NCHUNK * K, B)
        for k in range(K):
            m = jnp.min(sv, axis=0)
            eq = sv == m[None, :]
            idx = jnp.min(jnp.where(eq, si, BIGI), axis=0)
            sv = jnp.where(si == idx[None, :], BIGF, sv)
            base = (idx >> 7) * LB + (idx & 127)
            for g in range(GPS):
                cols_ref[k * GPS + g, :] = base + 128 * g


def _stage2(bm):
    return pl.pallas_call(
        _k2_body,
        grid=(NCHUNK,),
        in_specs=[pl.BlockSpec((B, CW), lambda c: (0, c))],
        out_specs=pl.BlockSpec((NCAND, B), lambda c: (0, 0)),
        out_shape=jax.ShapeDtypeStruct((NCAND, B), jnp.int32),
        scratch_shapes=[
            pltpu.VMEM((NCHUNK, K, B), jnp.float32),
            pltpu.VMEM((NCHUNK, K, B), jnp.int32),
        ],
    )(bm)


NG = NCAND * B
CHUNK = 128


def _sc_gather(leaf_pad, ids_pad, idx_flat):
    info = plsc.get_sparse_core_info()
    nc, ns = info.num_cores, info.num_subcores
    nw = nc * ns
    b_per_w = NG // nw
    nchunks = b_per_w // CHUNK
    mesh = plsc.VectorSubcoreMesh(core_axis_name="c", subcore_axis_name="s")

    @functools.partial(
        pl.kernel, mesh=mesh,
        compiler_params=pltpu.CompilerParams(use_tc_tiling_on_sc=False),
        out_type=[
            jax.ShapeDtypeStruct((NG, D), jnp.float32),
            jax.ShapeDtypeStruct((NG,), jnp.int32),
        ],
        scratch_types=[
            pltpu.VMEM((CHUNK,), jnp.int32),
            pltpu.VMEM((CHUNK, D), jnp.float32),
            pltpu.VMEM((CHUNK,), jnp.int32),
            pltpu.SemaphoreType.DMA,
            pltpu.SemaphoreType.DMA,
        ],
    )
    def k(table_hbm, ids_hbm, idx_hbm, rows_out, gids_out,
          idx_v, rows_v, gid_v, sem, sem2):
        wid = lax.axis_index("s") * nc + lax.axis_index("c")
        base0 = wid * b_per_w
        for ci in range(nchunks):
            base = base0 + ci * CHUNK
            pltpu.sync_copy(idx_hbm.at[pl.ds(base, CHUNK)], idx_v)
            pltpu.async_copy(table_hbm.at[idx_v], rows_v, sem).wait()
            pltpu.async_copy(ids_hbm.at[idx_v], gid_v, sem2).wait()
            pltpu.sync_copy(rows_v, rows_out.at[pl.ds(base, CHUNK)])
            pltpu.sync_copy(gid_v, gids_out.at[pl.ds(base, CHUNK)])

    return k(leaf_pad, ids_pad, idx_flat)


CB = 16
NCB = NCAND // CB


def _k4_body(z_ref, v_ref, gid_ref, col_ref, thr_ref,
             ood_ref, isood_ref, ids_ref, ar_s):
    i = pl.program_id(0)
    z = z_ref[...]
    v = v_ref[...]
    colsb = col_ref[...]
    x2 = jnp.sum(z * z, axis=1)[None, :]
    y2 = jnp.sum(v * v, axis=2)
    zb = z.astype(jnp.bfloat16).astype(jnp.float32)
    vb = v.astype(jnp.bfloat16).astype(jnp.float32)
    dt = jnp.sum(vb * zb[None, :, :], axis=2)
    sq = jnp.maximum(x2 + y2 - 2.0 * dt, 0.0)
    den = jnp.maximum((1.0 - x2) * (1.0 - y2), EPS)
    ar_s[i] = 1.0 + 2.0 * sq / den

    @pl.when(i == NCB - 1)
    def _():
        gids = gid_ref[...]
        ar = ar_s[...].reshape(NCAND, B)
        ar = jnp.where(colsb < L, ar, BIGF)
        for k in range(K):
            m = jnp.min(ar, axis=0)
            eq = ar == m[None, :]
            csel = jnp.min(jnp.where(eq, colsb, BIGI), axis=0)
            selm = colsb == csel[None, :]
            ids_ref[k, :] = jnp.sum(jnp.where(selm, gids, 0), axis=0)
            ar = jnp.where(selm, BIGF, ar)
            if k == 0:
                mc = jnp.maximum(m, 1.0 + EPS)
                ood = jnp.log(mc + jnp.sqrt((mc - 1.0) * (mc + 1.0)))
                ood_ref[0, :] = ood
                isood_ref[0, :] = (ood > thr_ref[0, 0]).astype(jnp.int32)


def _stage4(z, rows, gids, cols, thr):
    return pl.pallas_call(
        _k4_body,
        grid=(NCB,),
        in_specs=[
            pl.BlockSpec((B, D), lambda i: (0, 0)),
            pl.BlockSpec((CB, B, D), lambda i: (i, 0, 0)),
            pl.BlockSpec((NCAND, B), lambda i: (0, 0)),
            pl.BlockSpec((NCAND, B), lambda i: (0, 0)),
            pl.BlockSpec((1, 1), lambda i: (0, 0)),
        ],
        out_specs=[
            pl.BlockSpec((1, B), lambda i: (0, 0)),
            pl.BlockSpec((1, B), lambda i: (0, 0)),
            pl.BlockSpec((K, B), lambda i: (0, 0)),
        ],
        out_shape=[
            jax.ShapeDtypeStruct((1, B), jnp.float32),
            jax.ShapeDtypeStruct((1, B), jnp.int32),
            jax.ShapeDtypeStruct((K, B), jnp.int32),
        ],
        scratch_shapes=[pltpu.VMEM((NCB, CB, B), jnp.float32)],
    )(z, rows, gids, cols, thr)


def kernel(z_hyp, leaf_emb, leaf_node_ids, threshold):
    leaf_pad = jnp.pad(leaf_emb, ((0, LP - L), (0, 0)))
    ids_pad = jnp.pad(leaf_node_ids, (0, LP - L))
    bm = _stage1(z_hyp, leaf_pad)
    cols = _stage2(bm)
    rows, gids = _sc_gather(leaf_pad, ids_pad, cols.reshape(-1))
    ood, isood, ids = _stage4(
        z_hyp,
        rows.reshape(NCAND, B, D),
        gids.reshape(NCAND, B),
        cols,
        jnp.reshape(threshold, (1, 1)),
    )
    return (ood.reshape(B), isood.reshape(B).astype(bool), ids.T)


# speedup vs baseline: 5.6279x; 1.1816x over previous
import functools

import jax
import jax.numpy as jnp
from jax import lax
from jax.experimental import pallas as pl
from jax.experimental.pallas import tpu as pltpu
from jax.experimental.pallas import tpu_sc as plsc

EPS = 1e-7
B = 1024
L = 100000
D = 64
LB = 2048
NBLK = 49
LP = NBLK * LB
GPS = LB // 128
NSLOT = NBLK * 128
K = 5
NCAND = K * GPS
BIGV = 1e30
BIGF = 3e38
BIGI = 2 ** 30


NMAIN = L // LB
TAIL = L - NMAIN * LB
TAILP = 1792
TGPS = TAILP // 128


def _hi_lo(v):
    hi = v.astype(jnp.bfloat16).astype(jnp.float32)
    lo = (v - hi).astype(jnp.bfloat16).astype(jnp.float32)
    return hi, lo


def _proxy_min(z, x2h, x2l, ones, y, u):
    nl = y.shape[0]
    y2 = jnp.sum(y * y, axis=1)
    r = 1.0 / (1.0 - y2)
    y2h, y2l = _hi_lo(y2)
    onesy = jnp.ones((nl, 1), jnp.float32)
    ya = jnp.concatenate(
        [y * -2.0, onesy, onesy,
         jnp.reshape(y2h, (nl, 1)), jnp.reshape(y2l, (nl, 1))], axis=1)
    xa = jnp.concatenate([z, x2h, x2l, ones, ones], axis=1)
    sq = lax.dot_general(xa, ya, (((1,), (1,)), ((), ())),
                         preferred_element_type=jnp.float32)
    p = sq * r[None, :]
    if u is not None:
        p = p + u[None, :]
    m = p[:, 0:128]
    for g in range(1, p.shape[1] // 128):
        m = jnp.minimum(m, p[:, g * 128:(g + 1) * 128])
    return m


def _k1_body(z_ref, y_ref, t_ref, bm_ref):
    i = pl.program_id(0)
    z = z_ref[...]
    x2 = jnp.sum(z * z, axis=1)
    x2h, x2l = _hi_lo(x2)
    x2h = jnp.reshape(x2h, (B, 1))
    x2l = jnp.reshape(x2l, (B, 1))
    ones = jnp.ones((B, 1), jnp.float32)

    @pl.when(i < NMAIN)
    def _():
        bm_ref[...] = _proxy_min(z, x2h, x2l, ones, y_ref[...], None)

    @pl.when(i == NMAIN)
    def _():
        col = jax.lax.broadcasted_iota(jnp.int32, (1, TAILP), 1)[0]
        u = jnp.where(col < TAIL, 0.0, BIGV)
        bm_ref[...] = _proxy_min(z, x2h, x2l, ones, t_ref[...], u)


def _stage1(z, leaf_emb, tail_pad):
    return pl.pallas_call(
        _k1_body,
        grid=(NBLK,),
        in_specs=[
            pl.BlockSpec((B, D), lambda i: (0, 0)),
            pl.BlockSpec((LB, D), lambda i: (jnp.minimum(i, NMAIN - 1), 0)),
            pl.BlockSpec((TAILP, D), lambda i: (0, 0)),
        ],
        out_specs=pl.BlockSpec((B, 128), lambda i: (0, i)),
        out_shape=jax.ShapeDtypeStruct((B, NSLOT), jnp.float32),
    )(z, leaf_emb, tail_pad)


NCHUNK = 7
CW = NSLOT // NCHUNK


def _k2_body(bm_ref, cols_ref, colc_ref, sv_ref, si_ref):
    c = pl.program_id(0)
    v = bm_ref[...]
    sl = jax.lax.broadcasted_iota(jnp.int32, (B, CW), 1) + c * CW
    for k in range(K):
        m = jnp.min(v, axis=1)
        eq = v == m[:, None]
        idx = jnp.min(jnp.where(eq, sl, BIGI), axis=1)
        sv_ref[c, k, :] = m
        si_ref[c, k, :] = idx
        v = jnp.where(sl == idx[:, None], BIGF, v)

    @pl.when(c == NCHUNK - 1)
    def _():
        sv = sv_ref[...].reshape(NCHUNK * K, B)
        si = si_ref[...].reshape(NCHUNK * K, B)
        for k in range(K):
            m = jnp.min(sv, axis=0)
            eq = sv == m[None, :]
            idx = jnp.min(jnp.where(eq, si, BIGI), axis=0)
            sv = jnp.where(si == idx[None, :], BIGF, sv)
            base = (idx >> 7) * LB + (idx & 127)
            for g in range(GPS):
                c_raw = base + 128 * g
                cols_ref[k * GPS + g, :] = c_raw
                colc_ref[k * GPS + g, :] = jnp.minimum(c_raw, L - 1)


def _stage2(bm):
    return pl.pallas_call(
        _k2_body,
        grid=(NCHUNK,),
        in_specs=[pl.BlockSpec((B, CW), lambda c: (0, c))],
        out_specs=[
            pl.BlockSpec((NCAND, B), lambda c: (0, 0)),
            pl.BlockSpec((NCAND, B), lambda c: (0, 0)),
        ],
        out_shape=[
            jax.ShapeDtypeStruct((NCAND, B), jnp.int32),
            jax.ShapeDtypeStruct((NCAND, B), jnp.int32),
        ],
        scratch_shapes=[
            pltpu.VMEM((NCHUNK, K, B), jnp.float32),
            pltpu.VMEM((NCHUNK, K, B), jnp.int32),
        ],
    )(bm)


NG = NCAND * B
CHUNK = 128


def _sc_gather(leaf_pad, ids_pad, idx_flat):
    info = plsc.get_sparse_core_info()
    nc, ns = info.num_cores, info.num_subcores
    nw = nc * ns
    b_per_w = NG // nw
    nchunks = b_per_w // CHUNK
    mesh = plsc.VectorSubcoreMesh(core_axis_name="c", subcore_axis_name="s")

    @functools.partial(
        pl.kernel, mesh=mesh,
        compiler_params=pltpu.CompilerParams(use_tc_tiling_on_sc=False),
        out_type=[
            jax.ShapeDtypeStruct((NG, D), jnp.float32),
            jax.ShapeDtypeStruct((NG,), jnp.int32),
        ],
        scratch_types=[
            pltpu.VMEM((CHUNK,), jnp.int32),
            pltpu.VMEM((CHUNK, D), jnp.float32),
            pltpu.VMEM((CHUNK,), jnp.int32),
            pltpu.SemaphoreType.DMA,
            pltpu.SemaphoreType.DMA,
        ],
    )
    def k(table_hbm, ids_hbm, idx_hbm, rows_out, gids_out,
          idx_v, rows_v, gid_v, sem, sem2):
        wid = lax.axis_index("s") * nc + lax.axis_index("c")
        base0 = wid * b_per_w
        for ci in range(nchunks):
            base = base0 + ci * CHUNK
            pltpu.sync_copy(idx_hbm.at[pl.ds(base, CHUNK)], idx_v)
            crows = pltpu.async_copy(table_hbm.at[idx_v], rows_v, sem)
            cids = pltpu.async_copy(ids_hbm.at[idx_v], gid_v, sem2)
            crows.wait()
            cids.wait()
            pltpu.sync_copy(rows_v, rows_out.at[pl.ds(base, CHUNK)])
            pltpu.sync_copy(gid_v, gids_out.at[pl.ds(base, CHUNK)])

    return k(leaf_pad, ids_pad, idx_flat)


CB = 16
NCB = NCAND // CB


def _k4_body(z_ref, v_ref, gid_ref, col_ref, thr_ref,
             ood_ref, isood_ref, ids_ref, ar_s):
    i = pl.program_id(0)
    z = z_ref[...]
    v = v_ref[...]
    colsb = col_ref[...]
    x2 = jnp.sum(z * z, axis=1)[None, :]
    y2 = jnp.sum(v * v, axis=2)
    zb = z.astype(jnp.bfloat16).astype(jnp.float32)
    vb = v.astype(jnp.bfloat16).astype(jnp.float32)
    dt = jnp.sum(vb * zb[None, :, :], axis=2)
    sq = jnp.maximum(x2 + y2 - 2.0 * dt, 0.0)
    den = jnp.maximum((1.0 - x2) * (1.0 - y2), EPS)
    ar_s[i] = 1.0 + 2.0 * sq / den

    @pl.when(i == NCB - 1)
    def _():
        gids = gid_ref[...]
        ar = ar_s[...].reshape(NCAND, B)
        ar = jnp.where(colsb < L, ar, BIGF)
        for k in range(K):
            m = jnp.min(ar, axis=0)
            eq = ar == m[None, :]
            csel = jnp.min(jnp.where(eq, colsb, BIGI), axis=0)
            selm = colsb == csel[None, :]
            ids_ref[k, :] = jnp.sum(jnp.where(selm, gids, 0), axis=0)
            ar = jnp.where(selm, BIGF, ar)
            if k == 0:
                mc = jnp.maximum(m, 1.0 + EPS)
                ood = jnp.log(mc + jnp.sqrt((mc - 1.0) * (mc + 1.0)))
                ood_ref[0, :] = ood
                isood_ref[0, :] = (ood > thr_ref[0, 0]).astype(jnp.int32)


def _stage4(z, rows, gids, cols, thr):
    return pl.pallas_call(
        _k4_body,
        grid=(NCB,),
        in_specs=[
            pl.BlockSpec((B, D), lambda i: (0, 0)),
            pl.BlockSpec((CB, B, D), lambda i: (i, 0, 0)),
            pl.BlockSpec((NCAND, B), lambda i: (0, 0)),
            pl.BlockSpec((NCAND, B), lambda i: (0, 0)),
            pl.BlockSpec((1, 1), lambda i: (0, 0)),
        ],
        out_specs=[
            pl.BlockSpec((1, B), lambda i: (0, 0)),
            pl.BlockSpec((1, B), lambda i: (0, 0)),
            pl.BlockSpec((K, B), lambda i: (0, 0)),
        ],
        out_shape=[
            jax.ShapeDtypeStruct((1, B), jnp.float32),
            jax.ShapeDtypeStruct((1, B), jnp.int32),
            jax.ShapeDtypeStruct((K, B), jnp.int32),
        ],
        scratch_shapes=[pltpu.VMEM((NCB, CB, B), jnp.float32)],
    )(z, rows, gids, cols, thr)


def kernel(z_hyp, leaf_emb, leaf_node_ids, threshold):
    tail_pad = jnp.pad(leaf_emb[NMAIN * LB:], ((0, TAILP - TAIL), (0, 0)))
    bm = _stage1(z_hyp, leaf_emb, tail_pad)
    cols, colc = _stage2(bm)
    rows, gids = _sc_gather(leaf_emb, leaf_node_ids, colc.reshape(-1))
    ood, isood, ids = _stage4(
        z_hyp,
        rows.reshape(NCAND, B, D),
        gids.reshape(NCAND, B),
        cols,
        jnp.reshape(threshold, (1, 1)),
    )
    return (ood.reshape(B), isood.reshape(B).astype(bool), ids.T)


# speedup vs baseline: 5.8868x; 1.0460x over previous
import functools

import jax
import jax.numpy as jnp
from jax import lax
from jax.experimental import pallas as pl
from jax.experimental.pallas import tpu as pltpu
from jax.experimental.pallas import tpu_sc as plsc

EPS = 1e-7
B = 1024
L = 100000
D = 64
LB = 2048
NBLK = 49
LP = NBLK * LB
GPS = LB // 128
NSLOT = NBLK * 128
K = 5
NCAND = K * GPS
BIGV = 1e30
BIGF = 3e38
BIGI = 2 ** 30


NMAIN = L // LB
TAIL = L - NMAIN * LB
TAILP = 1792
TGPS = TAILP // 128


def _hi_lo(v):
    hi = v.astype(jnp.bfloat16).astype(jnp.float32)
    lo = (v - hi).astype(jnp.bfloat16).astype(jnp.float32)
    return hi, lo


def _proxy_min(z, x2h, x2l, ones, y, u):
    nl = y.shape[0]
    y2 = jnp.sum(y * y, axis=1)
    r = 1.0 / (1.0 - y2)
    y2h, y2l = _hi_lo(y2)
    onesy = jnp.ones((nl, 1), jnp.float32)
    ya = jnp.concatenate(
        [y * -2.0, onesy, onesy,
         jnp.reshape(y2h, (nl, 1)), jnp.reshape(y2l, (nl, 1))], axis=1)
    xa = jnp.concatenate([z, x2h, x2l, ones, ones], axis=1)
    sq = lax.dot_general(xa, ya, (((1,), (1,)), ((), ())),
                         preferred_element_type=jnp.float32)
    p = sq * r[None, :]
    if u is not None:
        p = p + u[None, :]
    m = p[:, 0:128]
    for g in range(1, p.shape[1] // 128):
        m = jnp.minimum(m, p[:, g * 128:(g + 1) * 128])
    return m


CBLK = 7


def _k12_body(z_ref, y_ref, t_ref, cols_ref, colc_ref, bms, sv_ref, si_ref):
    i = pl.program_id(0)
    z = z_ref[...]
    x2 = jnp.sum(z * z, axis=1)
    x2h, x2l = _hi_lo(x2)
    x2h = jnp.reshape(x2h, (B, 1))
    x2l = jnp.reshape(x2l, (B, 1))
    ones = jnp.ones((B, 1), jnp.float32)

    @pl.when(i < NMAIN)
    def _():
        bms[i % CBLK] = _proxy_min(z, x2h, x2l, ones, y_ref[...], None)

    @pl.when(i == NMAIN)
    def _():
        col = jax.lax.broadcasted_iota(jnp.int32, (1, TAILP), 1)[0]
        u = jnp.where(col < TAIL, 0.0, BIGV)
        bms[i % CBLK] = _proxy_min(z, x2h, x2l, ones, t_ref[...], u)

    @pl.when(i % CBLK == CBLK - 1)
    def _():
        c = i // CBLK
        v3 = bms[...]
        bi = jax.lax.broadcasted_iota(jnp.int32, (CBLK, B, 128), 0)
        li = jax.lax.broadcasted_iota(jnp.int32, (CBLK, B, 128), 2)
        sl3 = (c * CBLK + bi) * 128 + li
        for k in range(K):
            m = jnp.min(jnp.min(v3, axis=0), axis=1)
            eq = v3 == m[None, :, None]
            idx = jnp.min(jnp.min(jnp.where(eq, sl3, BIGI), axis=0), axis=1)
            sv_ref[c, k, :] = m
            si_ref[c, k, :] = idx
            v3 = jnp.where(sl3 == idx[None, :, None], BIGF, v3)

    @pl.when(i == NBLK - 1)
    def _():
        sv = sv_ref[...].reshape(NCHUNK * K, B)
        si = si_ref[...].reshape(NCHUNK * K, B)
        for k in range(K):
            m = jnp.min(sv, axis=0)
            eq = sv == m[None, :]
            idx = jnp.min(jnp.where(eq, si, BIGI), axis=0)
            sv = jnp.where(si == idx[None, :], BIGF, sv)
            base = (idx >> 7) * LB + (idx & 127)
            for g in range(GPS):
                c_raw = base + 128 * g
                cols_ref[k * GPS + g, :] = c_raw
                colc_ref[k * GPS + g, :] = jnp.minimum(c_raw, L - 1)


def _stage12(z, leaf_emb, tail_pad):
    return pl.pallas_call(
        _k12_body,
        grid=(NBLK,),
        in_specs=[
            pl.BlockSpec((B, D), lambda i: (0, 0)),
            pl.BlockSpec((LB, D), lambda i: (jnp.minimum(i, NMAIN - 1), 0)),
            pl.BlockSpec((TAILP, D), lambda i: (0, 0)),
        ],
        out_specs=[
            pl.BlockSpec((NCAND, B), lambda i: (0, 0)),
            pl.BlockSpec((NCAND, B), lambda i: (0, 0)),
        ],
        out_shape=[
            jax.ShapeDtypeStruct((NCAND, B), jnp.int32),
            jax.ShapeDtypeStruct((NCAND, B), jnp.int32),
        ],
        scratch_shapes=[
            pltpu.VMEM((CBLK, B, 128), jnp.float32),
            pltpu.VMEM((NCHUNK, K, B), jnp.float32),
            pltpu.VMEM((NCHUNK, K, B), jnp.int32),
        ],
    )(z, leaf_emb, tail_pad)


NCHUNK = 7
CW = NSLOT // NCHUNK


NG = NCAND * B
CHUNK = 128


def _sc_gather(leaf_pad, ids_pad, idx_flat):
    info = plsc.get_sparse_core_info()
    nc, ns = info.num_cores, info.num_subcores
    nw = nc * ns
    b_per_w = NG // nw
    nchunks = b_per_w // CHUNK
    mesh = plsc.VectorSubcoreMesh(core_axis_name="c", subcore_axis_name="s")

    @functools.partial(
        pl.kernel, mesh=mesh,
        compiler_params=pltpu.CompilerParams(use_tc_tiling_on_sc=False),
        out_type=[
            jax.ShapeDtypeStruct((NG, D), jnp.float32),
            jax.ShapeDtypeStruct((NG,), jnp.int32),
        ],
        scratch_types=[
            pltpu.VMEM((2, CHUNK), jnp.int32),
            pltpu.VMEM((2, CHUNK, D), jnp.float32),
            pltpu.VMEM((2, CHUNK), jnp.int32),
            pltpu.SemaphoreType.DMA,
            pltpu.SemaphoreType.DMA,
            pltpu.SemaphoreType.DMA,
            pltpu.SemaphoreType.DMA,
        ],
    )
    def k(table_hbm, ids_hbm, idx_hbm, rows_out, gids_out,
          idx_v, rows_v, gid_v, semr0, semr1, semi0, semi1):
        wid = lax.axis_index("s") * nc + lax.axis_index("c")
        base0 = wid * b_per_w
        semr = (semr0, semr1)
        semi = (semi0, semi1)
        pend = {}

        def prefetch(cj):
            nb = cj % 2
            base = base0 + cj * CHUNK
            pltpu.sync_copy(idx_hbm.at[pl.ds(base, CHUNK)], idx_v.at[nb])
            pend[cj] = (
                pltpu.async_copy(table_hbm.at[idx_v.at[nb]],
                                 rows_v.at[nb], semr[nb]),
                pltpu.async_copy(ids_hbm.at[idx_v.at[nb]],
                                 gid_v.at[nb], semi[nb]),
            )

        prefetch(0)
        for ci in range(nchunks):
            if ci + 1 < nchunks:
                prefetch(ci + 1)
            cur = ci % 2
            crows, cids = pend.pop(ci)
            crows.wait()
            cids.wait()
            base = base0 + ci * CHUNK
            pltpu.sync_copy(rows_v.at[cur], rows_out.at[pl.ds(base, CHUNK)])
            pltpu.sync_copy(gid_v.at[cur], gids_out.at[pl.ds(base, CHUNK)])

    return k(leaf_pad, ids_pad, idx_flat)


CB = 16
NCB = NCAND // CB


def _k4_body(z_ref, v_ref, gid_ref, col_ref, thr_ref,
             ood_ref, isood_ref, ids_ref, ar_s):
    i = pl.program_id(0)
    z = z_ref[...]
    v = v_ref[...]
    colsb = col_ref[...]
    x2 = jnp.sum(z * z, axis=1)[None, :]
    y2 = jnp.sum(v * v, axis=2)
    zb = z.astype(jnp.bfloat16).astype(jnp.float32)
    vb = v.astype(jnp.bfloat16).astype(jnp.float32)
    dt = jnp.sum(vb * zb[None, :, :], axis=2)
    sq = jnp.maximum(x2 + y2 - 2.0 * dt, 0.0)
    den = jnp.maximum((1.0 - x2) * (1.0 - y2), EPS)
    ar_s[i] = 1.0 + 2.0 * sq / den

    @pl.when(i == NCB - 1)
    def _():
        gids = gid_ref[...]
        ar = ar_s[...].reshape(NCAND, B)
        ar = jnp.where(colsb < L, ar, BIGF)
        for k in range(K):
            m = jnp.min(ar, axis=0)
            eq = ar == m[None, :]
            csel = jnp.min(jnp.where(eq, colsb, BIGI), axis=0)
            selm = colsb == csel[None, :]
            ids_ref[k, :] = jnp.sum(jnp.where(selm, gids, 0), axis=0)
            ar = jnp.where(selm, BIGF, ar)
            if k == 0:
                mc = jnp.maximum(m, 1.0 + EPS)
                ood = jnp.log(mc + jnp.sqrt((mc - 1.0) * (mc + 1.0)))
                ood_ref[0, :] = ood
                isood_ref[0, :] = (ood > thr_ref[0, 0]).astype(jnp.int32)


def _stage4(z, rows, gids, cols, thr):
    return pl.pallas_call(
        _k4_body,
        grid=(NCB,),
        in_specs=[
            pl.BlockSpec((B, D), lambda i: (0, 0)),
            pl.BlockSpec((CB, B, D), lambda i: (i, 0, 0)),
            pl.BlockSpec((NCAND, B), lambda i: (0, 0)),
            pl.BlockSpec((NCAND, B), lambda i: (0, 0)),
            pl.BlockSpec((1, 1), lambda i: (0, 0)),
        ],
        out_specs=[
            pl.BlockSpec((1, B), lambda i: (0, 0)),
            pl.BlockSpec((1, B), lambda i: (0, 0)),
            pl.BlockSpec((K, B), lambda i: (0, 0)),
        ],
        out_shape=[
            jax.ShapeDtypeStruct((1, B), jnp.float32),
            jax.ShapeDtypeStruct((1, B), jnp.int32),
            jax.ShapeDtypeStruct((K, B), jnp.int32),
        ],
        scratch_shapes=[pltpu.VMEM((NCB, CB, B), jnp.float32)],
    )(z, rows, gids, cols, thr)


def kernel(z_hyp, leaf_emb, leaf_node_ids, threshold):
    tail_pad = jnp.pad(leaf_emb[NMAIN * LB:], ((0, TAILP - TAIL), (0, 0)))
    cols, colc = _stage12(z_hyp, leaf_emb, tail_pad)
    rows, gids = _sc_gather(leaf_emb, leaf_node_ids, colc.reshape(-1))
    ood, isood, ids = _stage4(
        z_hyp,
        rows.reshape(NCAND, B, D),
        gids.reshape(NCAND, B),
        cols,
        jnp.reshape(threshold, (1, 1)),
    )
    return (ood.reshape(B), isood.reshape(B).astype(bool), ids.T)
